# baseline (speedup 1.0000x reference)
import functools

import jax
import jax.numpy as jnp
from jax.experimental import pallas as pl
from jax.experimental.pallas import tpu as pltpu

_T = 1024


def _main_body(x_ref, wc_ref, bc_ref, woa_ref, wob_ref, wp_ref, bpo_ref,
               o_ref, m_ref, *, single_chunk):
    M = wp_ref.shape[0]
    T = x_ref.shape[1]

    if not single_chunk:
        @pl.when(pl.program_id(1) == 0)
        def _():
            m_ref[...] = jnp.zeros_like(m_ref)

    xb = x_ref[0].astype(jnp.bfloat16)
    g = jnp.dot(xb, wc_ref[...], preferred_element_type=jnp.float32) + bc_ref[...]
    gT = jnp.transpose(g)
    rw = jax.nn.sigmoid(gT[:M])
    ww = jax.nn.sigmoid(gT[M:2 * M])
    nm = gT[2 * M:3 * M]

    a = 1.0 - ww
    b = ww * nm

    def shift(X, k, fill):
        if k >= T:
            return jnp.full((M, T), fill, jnp.float32)
        pad = jnp.full((M, k), fill, jnp.float32)
        return jnp.concatenate([pad, X[:, : T - k]], axis=1)

    A, Bv = a, b
    k = 1
    while k < T:
        A1, B1 = shift(A, k, 1.0), shift(Bv, k, 0.0)
        A2, B2 = shift(A, 2 * k, 1.0), shift(Bv, 2 * k, 0.0)
        A3, B3 = shift(A, 3 * k, 1.0), shift(Bv, 3 * k, 0.0)
        Bv = Bv + A * (B1 + A1 * (B2 + A2 * B3))
        A = A * (A1 * (A2 * A3))
        k *= 4

    B_prev = jnp.concatenate([jnp.zeros((M, 1), jnp.float32), Bv[:, : T - 1]], axis=1)
    if single_chunk:
        rmT = rw * B_prev
    else:
        m_in = m_ref[:, :1]
        A_prev = jnp.concatenate(
            [jnp.ones((M, 1), jnp.float32), A[:, : T - 1]], axis=1)
        rmT = rw * (A_prev * m_in + B_prev)
        m_ref[:, :1] = A[:, T - 1:] * m_in + Bv[:, T - 1:]

    wob = wob_ref[...].astype(jnp.bfloat16)
    wq = jnp.dot(wp_ref[...].astype(jnp.bfloat16), wob,
                 preferred_element_type=jnp.float32).astype(jnp.bfloat16)
    bq = (jnp.dot(bpo_ref[:1].astype(jnp.bfloat16), wob,
                  preferred_element_type=jnp.float32) + bpo_ref[1:])

    h = (jnp.dot(xb, woa_ref[...].astype(jnp.bfloat16),
                 preferred_element_type=jnp.float32)
         + jnp.einsum("mt,md->td", rmT.astype(jnp.bfloat16), wq,
                      preferred_element_type=jnp.float32))
    o_ref[0] = jnp.tanh(h + bq)


@functools.partial(jax.jit, static_argnames=("interpret",))
def _run(x, Wr, br, Ww, bw, Wt, bt, Wp, bp, Wo, bo, interpret=False):
    B, S, D = x.shape
    M = Wr.shape[1]
    T = _T

    Wc = jnp.concatenate(
        [Wr, Ww, Wt, jnp.zeros((D, M), jnp.float32)], axis=1
    ).astype(jnp.bfloat16)
    bc = jnp.concatenate([br, bw, bt, jnp.zeros((M,), jnp.float32)]).reshape(1, 4 * M)
    bpo = jnp.stack([bp, bo])

    out = pl.pallas_call(
        functools.partial(_main_body, single_chunk=(T == S)),
        out_shape=jax.ShapeDtypeStruct((B, S, D), jnp.float32),
        grid=(B, S // T),
        in_specs=[
            pl.BlockSpec((1, T, D), lambda bi, ti: (bi, ti, 0)),
            pl.BlockSpec((D, 4 * M), lambda bi, ti: (0, 0)),
            pl.BlockSpec((1, 4 * M), lambda bi, ti: (0, 0)),
            pl.BlockSpec((D, D), lambda bi, ti: (0, 0)),
            pl.BlockSpec((D, D), lambda bi, ti: (1, 0)),
            pl.BlockSpec((M, D), lambda bi, ti: (0, 0)),
            pl.BlockSpec((2, D), lambda bi, ti: (0, 0)),
        ],
        out_specs=pl.BlockSpec((1, T, D), lambda bi, ti: (bi, ti, 0)),
        scratch_shapes=[pltpu.VMEM((M, 128), jnp.float32)],
        compiler_params=pltpu.CompilerParams(
            dimension_semantics=("parallel", "arbitrary"),
            fuse_transposed_lhs_in_matmul=True,
        ),
        interpret=interpret,
    )(x, Wc, bc, Wo, Wo, Wp, bpo)
    return out


def kernel(x, Wr, br, Ww, bw, Wt, bt, Wp, bp, Wo, bo):
    return _run(x, Wr, br, Ww, bw, Wt, bt, Wp, bp, Wo, bo)


# speedup vs baseline: 1.0098x; 1.0098x over previous
import functools

import jax
import jax.numpy as jnp
from jax.experimental import pallas as pl
from jax.experimental.pallas import tpu as pltpu

_T = 2048


def _main_body(x_ref, wc_ref, bc_ref, woa_ref, wob_ref, wp_ref, bpo_ref,
               o_ref, m_ref, *, single_chunk):
    M = wp_ref.shape[0]
    T = x_ref.shape[1]

    if not single_chunk:
        @pl.when(pl.program_id(1) == 0)
        def _():
            m_ref[...] = jnp.zeros_like(m_ref)

    xb = x_ref[0].astype(jnp.bfloat16)
    g = jnp.dot(xb, wc_ref[...], preferred_element_type=jnp.float32) + bc_ref[...]
    gT = jnp.transpose(g)
    rw = jax.nn.sigmoid(gT[:M])
    ww = jax.nn.sigmoid(gT[M:2 * M])
    nm = gT[2 * M:3 * M]

    a = 1.0 - ww
    b = ww * nm

    def shift(X, k, fill):
        if k >= T:
            return jnp.full((M, T), fill, jnp.float32)
        pad = jnp.full((M, k), fill, jnp.float32)
        return jnp.concatenate([pad, X[:, : T - k]], axis=1)

    A, Bv = a, b
    k = 1
    while k < T:
        A1, B1 = shift(A, k, 1.0), shift(Bv, k, 0.0)
        A2, B2 = shift(A, 2 * k, 1.0), shift(Bv, 2 * k, 0.0)
        A3, B3 = shift(A, 3 * k, 1.0), shift(Bv, 3 * k, 0.0)
        Bv = Bv + A * (B1 + A1 * (B2 + A2 * B3))
        A = A * (A1 * (A2 * A3))
        k *= 4

    B_prev = jnp.concatenate([jnp.zeros((M, 1), jnp.float32), Bv[:, : T - 1]], axis=1)
    if single_chunk:
        rmT = rw * B_prev
    else:
        m_in = m_ref[:, :1]
        A_prev = jnp.concatenate(
            [jnp.ones((M, 1), jnp.float32), A[:, : T - 1]], axis=1)
        rmT = rw * (A_prev * m_in + B_prev)
        m_ref[:, :1] = A[:, T - 1:] * m_in + Bv[:, T - 1:]

    wob = wob_ref[...].astype(jnp.bfloat16)
    wq = jnp.dot(wp_ref[...].astype(jnp.bfloat16), wob,
                 preferred_element_type=jnp.float32).astype(jnp.bfloat16)
    bq = (jnp.dot(bpo_ref[:1].astype(jnp.bfloat16), wob,
                  preferred_element_type=jnp.float32) + bpo_ref[1:])

    h = (jnp.dot(xb, woa_ref[...].astype(jnp.bfloat16),
                 preferred_element_type=jnp.float32)
         + jnp.einsum("mt,md->td", rmT.astype(jnp.bfloat16), wq,
                      preferred_element_type=jnp.float32))
    o_ref[0] = jnp.tanh(h + bq)


@functools.partial(jax.jit, static_argnames=("interpret",))
def _run(x, Wr, br, Ww, bw, Wt, bt, Wp, bp, Wo, bo, interpret=False):
    B, S, D = x.shape
    M = Wr.shape[1]
    T = _T

    Wc = jnp.concatenate(
        [Wr, Ww, Wt, jnp.zeros((D, M), jnp.float32)], axis=1
    ).astype(jnp.bfloat16)
    bc = jnp.concatenate([br, bw, bt, jnp.zeros((M,), jnp.float32)]).reshape(1, 4 * M)
    bpo = jnp.stack([bp, bo])

    out = pl.pallas_call(
        functools.partial(_main_body, single_chunk=(T == S)),
        out_shape=jax.ShapeDtypeStruct((B, S, D), jnp.float32),
        grid=(B, S // T),
        in_specs=[
            pl.BlockSpec((1, T, D), lambda bi, ti: (bi, ti, 0)),
            pl.BlockSpec((D, 4 * M), lambda bi, ti: (0, 0)),
            pl.BlockSpec((1, 4 * M), lambda bi, ti: (0, 0)),
            pl.BlockSpec((D, D), lambda bi, ti: (0, 0)),
            pl.BlockSpec((D, D), lambda bi, ti: (1, 0)),
            pl.BlockSpec((M, D), lambda bi, ti: (0, 0)),
            pl.BlockSpec((2, D), lambda bi, ti: (0, 0)),
        ],
        out_specs=pl.BlockSpec((1, T, D), lambda bi, ti: (bi, ti, 0)),
        scratch_shapes=[pltpu.VMEM((M, 128), jnp.float32)],
        compiler_params=pltpu.CompilerParams(
            dimension_semantics=("parallel", "arbitrary"),
            fuse_transposed_lhs_in_matmul=True,
        ),
        interpret=interpret,
    )(x, Wc, bc, Wo, Wo, Wp, bpo)
    return out


def kernel(x, Wr, br, Ww, bw, Wt, bt, Wp, bp, Wo, bo):
    return _run(x, Wr, br, Ww, bw, Wt, bt, Wp, bp, Wo, bo)
